# baseline (speedup 1.0000x reference)
import functools

import jax
import jax.numpy as jnp
from jax.experimental import pallas as pl
from jax.experimental.pallas import tpu as pltpu

_BH = 512
_CH = 8


def _h23(a):
    left = jnp.concatenate([a[:, :1], a[:, :-1]], axis=1)
    right = jnp.concatenate([a[:, 1:], a[:, -1:]], axis=1)
    h2 = jnp.maximum(left, right)
    return h2, jnp.maximum(h2, a)


def _nms_body(bh, ch, x_ref, bot_ref, o_ref, tc_ref):
    i = pl.program_id(1)
    ni = pl.num_programs(1)

    def row_h3(r):
        _, h3 = _h23(r)
        return h3

    def emit(a, h2, h3, above_row, below_row):
        above = jnp.concatenate([above_row, h3[:-1]], axis=0)
        below = jnp.concatenate([h3[1:], below_row], axis=0)
        nm = jnp.maximum(jnp.maximum(above, below), h2)
        c2 = x_ref[0, a : a + ch, :]
        o_ref[0, a : a + ch, :] = jnp.where(c2 > nm, c2, 0.0)

    state = None
    for c in range(bh // ch):
        a = c * ch
        cur = x_ref[0, a : a + ch, :]
        h2, h3 = _h23(cur)
        if c == 0:
            above_row = jnp.where(i == 0, h3[0:1], tc_ref[0:1, :])
        else:
            pa, ph2, ph3, prow = state
            emit(pa, ph2, ph3, prow, h3[0:1])
            above_row = ph3[ch - 1 : ch]
        state = (a, h2, h3, above_row)
    pa, ph2, ph3, prow = state
    below_row = jnp.where(
        i == ni - 1, ph3[ch - 1 : ch], row_h3(bot_ref[0, 0:1, :])
    )
    emit(pa, ph2, ph3, prow, below_row)
    tc_ref[0:1, :] = ph3[ch - 1 : ch]


def _nms(x, *, interpret=False):
    b, c, h, w = x.shape
    bc = b * c
    xr = x.reshape(bc, h, w)
    bh = min(_BH, h)
    ni = h // bh
    ch = min(_CH, bh)
    g8 = h // 8
    bh8 = bh // 8

    out = pl.pallas_call(
        functools.partial(_nms_body, bh, ch),
        out_shape=jax.ShapeDtypeStruct((bc, h, w), x.dtype),
        grid=(bc, ni),
        in_specs=[
            pl.BlockSpec((1, bh, w), lambda b_, i: (b_, i, 0)),
            pl.BlockSpec(
                (1, 8, w),
                lambda b_, i: (b_, jnp.minimum((i + 1) * bh8, g8 - 1), 0),
            ),
        ],
        out_specs=pl.BlockSpec((1, bh, w), lambda b_, i: (b_, i, 0)),
        scratch_shapes=[pltpu.VMEM((8, w), jnp.float32)],
        compiler_params=pltpu.CompilerParams(
            dimension_semantics=("parallel", "arbitrary"),
            vmem_limit_bytes=52 * 1024 * 1024,
        ),
        name="nms2d",
        interpret=interpret,
    )(xr, xr)
    return out.reshape(b, c, h, w)


def kernel(x):
    return _nms(x)


# speedup vs baseline: 1.0999x; 1.0999x over previous
import functools

import jax
import jax.numpy as jnp
from jax.experimental import pallas as pl
from jax.experimental.pallas import tpu as pltpu

_BH = 1024
_CH = 8


def _h23(a):
    left = jnp.concatenate([a[:, :1], a[:, :-1]], axis=1)
    right = jnp.concatenate([a[:, 1:], a[:, -1:]], axis=1)
    h2 = jnp.maximum(left, right)
    return h2, jnp.maximum(h2, a)


def _nms_body(bh, ch, x_ref, bot_ref, o_ref, tc_ref):
    i = pl.program_id(1)
    ni = pl.num_programs(1)

    def row_h3(r):
        _, h3 = _h23(r)
        return h3

    def emit(a, h2, h3, above_row, below_row):
        above = jnp.concatenate([above_row, h3[:-1]], axis=0)
        below = jnp.concatenate([h3[1:], below_row], axis=0)
        nm = jnp.maximum(jnp.maximum(above, below), h2)
        c2 = x_ref[0, a : a + ch, :]
        o_ref[0, a : a + ch, :] = jnp.where(c2 > nm, c2, 0.0)

    state = None
    for c in range(bh // ch):
        a = c * ch
        cur = x_ref[0, a : a + ch, :]
        h2, h3 = _h23(cur)
        if c == 0:
            above_row = jnp.where(i == 0, h3[0:1], tc_ref[0:1, :])
        else:
            pa, ph2, ph3, prow = state
            emit(pa, ph2, ph3, prow, h3[0:1])
            above_row = ph3[ch - 1 : ch]
        state = (a, h2, h3, above_row)
    pa, ph2, ph3, prow = state
    below_row = jnp.where(
        i == ni - 1, ph3[ch - 1 : ch], row_h3(bot_ref[0, 0:1, :])
    )
    emit(pa, ph2, ph3, prow, below_row)
    tc_ref[0:1, :] = ph3[ch - 1 : ch]


def _nms(x, *, interpret=False):
    b, c, h, w = x.shape
    bc = b * c
    xr = x.reshape(bc, h, w)
    bh = min(_BH, h)
    ni = h // bh
    ch = min(_CH, bh)
    g8 = h // 8
    bh8 = bh // 8

    out = pl.pallas_call(
        functools.partial(_nms_body, bh, ch),
        out_shape=jax.ShapeDtypeStruct((bc, h, w), x.dtype),
        grid=(bc, ni),
        in_specs=[
            pl.BlockSpec((1, bh, w), lambda b_, i: (b_, i, 0)),
            pl.BlockSpec(
                (1, 8, w),
                lambda b_, i: (b_, jnp.minimum((i + 1) * bh8, g8 - 1), 0),
            ),
        ],
        out_specs=pl.BlockSpec((1, bh, w), lambda b_, i: (b_, i, 0)),
        scratch_shapes=[pltpu.VMEM((8, w), jnp.float32)],
        compiler_params=pltpu.CompilerParams(
            dimension_semantics=("parallel", "arbitrary"),
            vmem_limit_bytes=52 * 1024 * 1024,
        ),
        name="nms2d",
        interpret=interpret,
    )(xr, xr)
    return out.reshape(b, c, h, w)


def kernel(x):
    return _nms(x)
